# baseline (speedup 1.0000x reference)
import jax
import jax.numpy as jnp
from jax.experimental import pallas as pl
from jax.experimental.pallas import tpu as pltpu


def _round_up(x: int, m: int) -> int:
    return ((x + m - 1) // m) * m


def _fc_kernel(x_any, w_ref, b_ref, out1_ref, out3_ref, xbuf, lsem, osem):
    c = pl.program_id(0)
    j = pl.program_id(1)
    g = pl.num_programs(1)
    half = xbuf.shape[0]
    tm = out3_ref.shape[0]
    row0 = c * half
    load = pltpu.make_async_copy(
        x_any.at[pl.ds(row0, half), :], xbuf, lsem)
    store = pltpu.make_async_copy(
        xbuf, out1_ref.at[pl.ds(row0, half), :], osem)

    @pl.when(j == 0)
    def _():
        load.start()
        load.wait()
        store.start()

    x = xbuf[pl.ds(j * tm, tm), :].astype(jnp.bfloat16)
    w = w_ref[...].astype(jnp.bfloat16)
    acc = jax.lax.dot_general(
        x, w, dimension_numbers=(((1,), (1,)), ((), ())),
        preferred_element_type=jnp.float32)
    out3_ref[...] = (acc + b_ref[...]).astype(out3_ref.dtype)

    @pl.when(j == g - 1)
    def _():
        store.wait()


@jax.jit
def kernel(x_nchw, weight, bias):
    n = x_nchw.shape[0]
    x_flat = jnp.reshape(x_nchw, (n, -1))
    num_ftrs = x_flat.shape[1]
    num_classes = weight.shape[0]
    out_dtype = x_flat.dtype

    b2d = bias.astype(jnp.float32).reshape(1, num_classes)

    tm = 512
    n_pad = _round_up(n, 2 * tm)
    x_p = x_flat if n_pad == n else jnp.pad(x_flat, ((0, n_pad - n), (0, 0)))
    g = n_pad // tm // 2
    half = g * tm

    out1_p, out3_p = pl.pallas_call(
        _fc_kernel,
        out_shape=(
            jax.ShapeDtypeStruct((n_pad, num_ftrs), out_dtype),
            jax.ShapeDtypeStruct((n_pad, num_classes), out_dtype),
        ),
        grid=(2, g),
        in_specs=[
            pl.BlockSpec(memory_space=pl.ANY),
            pl.BlockSpec((num_classes, num_ftrs), lambda c, j: (0, 0)),
            pl.BlockSpec((1, num_classes), lambda c, j: (0, 0)),
        ],
        out_specs=(
            pl.BlockSpec(memory_space=pl.ANY),
            pl.BlockSpec((tm, num_classes), lambda c, j: (c * g + j, 0)),
        ),
        scratch_shapes=[
            pltpu.VMEM((half, num_ftrs), jnp.float32),
            pltpu.SemaphoreType.DMA,
            pltpu.SemaphoreType.DMA,
        ],
        compiler_params=pltpu.CompilerParams(
            dimension_semantics=("parallel", "arbitrary"),
            vmem_limit_bytes=60 * 1024 * 1024,
        ),
    )(x_p, weight, b2d)

    if n_pad == n:
        return out1_p, out3_p
    return out1_p[:n], out3_p[:n]


# speedup vs baseline: 1.1946x; 1.1946x over previous
import jax
import jax.numpy as jnp
from jax.experimental import pallas as pl
from jax.experimental.pallas import tpu as pltpu


def _round_up(x: int, m: int) -> int:
    return ((x + m - 1) // m) * m


def _fc_nt_kernel(x_ref, w_ref, b_ref, out_ref):
    x = x_ref[...].astype(jnp.bfloat16)
    w = w_ref[...].astype(jnp.bfloat16)
    acc = jax.lax.dot_general(
        x, w, dimension_numbers=(((1,), (1,)), ((), ())),
        preferred_element_type=jnp.float32)
    out_ref[...] = (acc + b_ref[...]).astype(out_ref.dtype)


@jax.jit
def kernel(x_nchw, weight, bias):
    n = x_nchw.shape[0]
    x_flat = jnp.reshape(x_nchw, (n, -1))
    num_ftrs = x_flat.shape[1]
    num_classes = weight.shape[0]
    out_dtype = x_flat.dtype

    b2d = bias.astype(jnp.float32).reshape(1, num_classes)

    tm = min(1024, _round_up(max(n, 8), 8))
    n_pad = _round_up(n, tm)
    x_p = x_flat if n_pad == n else jnp.pad(x_flat, ((0, n_pad - n), (0, 0)))

    out3_p = pl.pallas_call(
        _fc_nt_kernel,
        out_shape=jax.ShapeDtypeStruct((n_pad, num_classes), out_dtype),
        grid=(n_pad // tm,),
        in_specs=[
            pl.BlockSpec((tm, num_ftrs), lambda i: (i, 0)),
            pl.BlockSpec((num_classes, num_ftrs), lambda i: (0, 0)),
            pl.BlockSpec((1, num_classes), lambda i: (0, 0)),
        ],
        out_specs=pl.BlockSpec((tm, num_classes), lambda i: (i, 0)),
        compiler_params=pltpu.CompilerParams(
            dimension_semantics=("parallel",),
            vmem_limit_bytes=48 * 1024 * 1024,
        ),
    )(x_p, weight, b2d)

    out1 = jnp.copy(x_flat)
    if n_pad == n:
        return out1, out3_p
    return out1, out3_p[:n]
